# baseline (speedup 1.0000x reference)
import functools

import jax
import jax.numpy as jnp
from jax import lax
from jax.experimental import pallas as pl
from jax.experimental.pallas import tpu as pltpu
from jax.experimental.pallas import tpu_sc as plsc

N = 10000
NP = 10240
D = 64
F_IN = 10
B = 4
K = 20
KP = 32
BLK = 256
NBLK = NP // BLK
NW = 32
PW = NP // NW
WIN = 16
NWIN = (PW // WIN) * B
ROWS = (K + 1) * WIN
TOT = B * N


def _topk_body(rows_ref, w_ref, nrm_ref, nrmc_ref, out_ref):
    rows = rows_ref[...]
    w = w_ref[...]
    dots = lax.dot_general(rows, w, (((1,), (1,)), ((), ())),
                           preferred_element_type=jnp.float32)
    nrm = nrm_ref[...]
    nrm_rows = nrmc_ref[...]
    cos = dots / (nrm_rows * nrm)
    ci = lax.broadcasted_iota(jnp.int32, (BLK, NP), 1)
    cos = jnp.where(ci < N, cos, -3.0)
    ri = lax.broadcasted_iota(jnp.int32, (BLK, 1), 0) + pl.program_id(0) * BLK
    valid_row = ri < N
    idxs = []
    big = jnp.int32(2**30)
    for _ in range(K):
        m = jnp.max(cos, axis=1, keepdims=True)
        idx = jnp.min(jnp.where(cos >= m, ci, big), axis=1, keepdims=True)
        idxs.append(jnp.where(valid_row, idx, ri))
        cos = jnp.where(ci == idx, -3.0, cos)
    idxs.extend([ri] * (KP - K))
    out_ref[...] = jnp.concatenate(idxs, axis=1)


def _topk32(w_emb_p):
    nrm = jnp.sqrt(jnp.sum(w_emb_p * w_emb_p, axis=1))
    return pl.pallas_call(
        _topk_body,
        grid=(NBLK,),
        in_specs=[
            pl.BlockSpec((BLK, D), lambda i: (i, 0)),
            pl.BlockSpec((NP, D), lambda i: (0, 0)),
            pl.BlockSpec((1, NP), lambda i: (0, 0)),
            pl.BlockSpec((BLK, 1), lambda i: (i, 0)),
        ],
        out_specs=pl.BlockSpec((BLK, KP), lambda i: (i, 0)),
        out_shape=jax.ShapeDtypeStruct((NP, KP), jnp.int32),
    )(w_emb_p, w_emb_p, nrm[None, :], nrm[:, None])


def _prep_body(x_ref, lw_ref, emb_ref, aj_ref, aemj_ref, ai_ref, aemi_ref,
               xl_ref, s_ref, d_ref):
    x = x_ref[0]
    xl = lax.dot_general(x, lw_ref[...], (((1,), (0,)), ((), ())),
                         preferred_element_type=jnp.float32)
    emb = emb_ref[...]
    xl_ref[0] = xl
    s_ref[...] = jnp.sum(xl * aj_ref[...] + emb * aemj_ref[...], axis=1,
                         keepdims=True)
    d_ref[...] = jnp.sum(xl * ai_ref[...] + emb * aemi_ref[...], axis=1,
                         keepdims=True)


def _prep(data_p, lin_W, w_emb_p, att_j, att_em_j, att_i, att_em_i):
    return pl.pallas_call(
        _prep_body,
        grid=(B, NBLK),
        in_specs=[
            pl.BlockSpec((1, BLK, F_IN), lambda b, j: (b, j, 0)),
            pl.BlockSpec((F_IN, D), lambda b, j: (0, 0)),
            pl.BlockSpec((BLK, D), lambda b, j: (j, 0)),
            pl.BlockSpec((1, D), lambda b, j: (0, 0)),
            pl.BlockSpec((1, D), lambda b, j: (0, 0)),
            pl.BlockSpec((1, D), lambda b, j: (0, 0)),
            pl.BlockSpec((1, D), lambda b, j: (0, 0)),
        ],
        out_specs=[
            pl.BlockSpec((1, BLK, D), lambda b, j: (b, j, 0)),
            pl.BlockSpec((BLK, 1), lambda b, j: (b * NBLK + j, 0)),
            pl.BlockSpec((BLK, 1), lambda b, j: (b * NBLK + j, 0)),
        ],
        out_shape=[
            jax.ShapeDtypeStruct((B, NP, D), jnp.float32),
            jax.ShapeDtypeStruct((B * NP, 1), jnp.float32),
            jax.ShapeDtypeStruct((B * NP, 1), jnp.float32),
        ],
    )(data_p, lin_W, w_emb_p, att_j[None, :], att_em_j[None, :],
      att_i[None, :], att_em_i[None, :])


def _gnn_sc_body(tk_hbm, xl_hbm, s_hbm, d_hbm, out_hbm,
                 s_tab, d_tab, tk_v, gidx, grows, wbuf, obuf, sem):
    cid = lax.axis_index("c")
    sid = lax.axis_index("s")
    wid = cid * 16 + sid
    base = wid * PW
    pltpu.sync_copy(s_hbm, s_tab)
    pltpu.sync_copy(d_hbm, d_tab)
    lane = lax.broadcasted_iota(jnp.int32, (WIN,), 0)

    def window(t, carry):
        b = t // (PW // WIN)
        win = t % (PW // WIN)
        node0 = base + win * WIN
        pltpu.sync_copy(tk_hbm.at[pl.ds(node0 * KP, WIN * KP)], tk_v)
        nodes = node0 + lane
        boff = b * NP
        cols = []
        for k in range(K):
            ck = plsc.load_gather(tk_v, [lane * KP + k])
            cols.append(ck)
            gidx[pl.ds(k * WIN, WIN)] = ck + boff
        gidx[pl.ds(K * WIN, WIN)] = nodes + boff
        c1 = pltpu.async_copy(xl_hbm.at[gidx.at[pl.ds(0, 112)]],
                              grows.at[pl.ds(0, 112)], sem)
        c2 = pltpu.async_copy(xl_hbm.at[gidx.at[pl.ds(112, 112)]],
                              grows.at[pl.ds(112, 112)], sem)
        c3 = pltpu.async_copy(xl_hbm.at[gidx.at[pl.ds(224, 112)]],
                              grows.at[pl.ds(224, 112)], sem)
        bvec = jnp.full((WIN,), 0, jnp.int32) + b
        d_vec = plsc.load_gather(d_tab, [bvec, nodes])
        s_own = plsc.load_gather(s_tab, [bvec, nodes])
        z = d_vec + s_own
        l_self = jnp.maximum(z, 0.2 * z)
        m = l_self
        for k in range(K):
            sk = plsc.load_gather(s_tab, [bvec, cols[k]])
            zk = d_vec + sk
            lk = jnp.maximum(zk, 0.2 * zk)
            lk = jnp.where(cols[k] == nodes, -1e30, lk)
            wbuf[pl.ds(k * WIN, WIN)] = lk
            m = jnp.maximum(m, lk)
        ssum = jnp.exp(l_self - m)
        wbuf[pl.ds(K * WIN, WIN)] = ssum
        for k in range(K):
            ek = jnp.exp(wbuf[pl.ds(k * WIN, WIN)] - m)
            wbuf[pl.ds(k * WIN, WIN)] = ek
            ssum = ssum + ek
        inv = 1.0 / (ssum + 1e-16)
        c1.wait()
        c2.wait()
        c3.wait()
        wv = [wbuf[pl.ds(k * WIN, WIN)] for k in range(K + 1)]

        def dim_step(dd, c):
            dvec = jnp.full((WIN,), 0, jnp.int32) + dd
            acc = wv[0] * plsc.load_gather(grows, [lane, dvec])
            for k in range(1, K + 1):
                acc = acc + wv[k] * plsc.load_gather(grows,
                                                     [k * WIN + lane, dvec])
            plsc.store_scatter(obuf, [lane, dvec], acc * inv)
            return c

        lax.fori_loop(0, D, dim_step, 0)
        pltpu.sync_copy(obuf, out_hbm.at[pl.ds(boff + node0, WIN), :])
        return carry

    lax.fori_loop(0, NWIN, window, 0)


def _gnn_sc(tk_flat, xl_flat, s2, d2):
    mesh = plsc.VectorSubcoreMesh(core_axis_name="c", subcore_axis_name="s")
    run = pl.kernel(
        _gnn_sc_body,
        out_type=jax.ShapeDtypeStruct((B * NP, D), jnp.float32),
        mesh=mesh,
        compiler_params=pltpu.CompilerParams(needs_layout_passes=False,
                                             use_tc_tiling_on_sc=False),
        scratch_types=[
            pltpu.VMEM((B, NP), jnp.float32),
            pltpu.VMEM((B, NP), jnp.float32),
            pltpu.VMEM((WIN * KP,), jnp.int32),
            pltpu.VMEM((ROWS,), jnp.int32),
            pltpu.VMEM((ROWS, D), jnp.float32),
            pltpu.VMEM((ROWS,), jnp.float32),
            pltpu.VMEM((WIN, D), jnp.float32),
            pltpu.SemaphoreType.DMA,
        ],
    )
    return run(tk_flat, xl_flat, s2, d2)


def _sums_body(g_ref, s1_ref, s2_ref):
    @pl.when(pl.program_id(0) == 0)
    def _():
        s1_ref[...] = jnp.zeros_like(s1_ref)
        s2_ref[...] = jnp.zeros_like(s2_ref)
    g = g_ref[...]
    s1_ref[...] += jnp.sum(g, axis=0, keepdims=True)
    s2_ref[...] += jnp.sum(g * g, axis=0, keepdims=True)


def _sums(g):
    return pl.pallas_call(
        _sums_body,
        grid=(B * NBLK,),
        in_specs=[pl.BlockSpec((BLK, D), lambda i: (i, 0))],
        out_specs=[pl.BlockSpec((1, D), lambda i: (0, 0)),
                   pl.BlockSpec((1, D), lambda i: (0, 0))],
        out_shape=[jax.ShapeDtypeStruct((1, D), jnp.float32),
                   jax.ShapeDtypeStruct((1, D), jnp.float32)],
    )(g)


def _bn1_body(g_ref, s1_ref, s2_ref, emb_ref, bias_ref, gam_ref, bet_ref,
              outm_ref, t1_ref, t2_ref):
    b = pl.program_id(0)
    j = pl.program_id(1)

    @pl.when(jnp.logical_and(b == 0, j == 0))
    def _():
        t1_ref[...] = jnp.zeros_like(t1_ref)
        t2_ref[...] = jnp.zeros_like(t2_ref)

    bias = bias_ref[...]
    s1 = s1_ref[...] + TOT * bias
    s2 = s2_ref[...] + 2.0 * bias * s1_ref[...] + TOT * bias * bias
    mu = s1 / TOT
    var = s2 / TOT - mu * mu
    x = g_ref[...] + bias
    y = (x - mu) / jnp.sqrt(var + 1e-5) * gam_ref[...] + bet_ref[...]
    y = jnp.maximum(y, 0.0)
    outm = y * emb_ref[...]
    outm_ref[...] = outm
    t1_ref[...] += jnp.sum(outm, axis=0, keepdims=True)
    t2_ref[...] += jnp.sum(outm * outm, axis=0, keepdims=True)


def _bn1(g, w_emb_p, gnn_bias, bn1_gamma, bn1_beta):
    return pl.pallas_call(
        _bn1_body,
        grid=(B, NBLK),
        in_specs=[
            pl.BlockSpec((BLK, D), lambda b, j: (b * NBLK + j, 0)),
            pl.BlockSpec((1, D), lambda b, j: (0, 0)),
            pl.BlockSpec((1, D), lambda b, j: (0, 0)),
            pl.BlockSpec((BLK, D), lambda b, j: (j, 0)),
            pl.BlockSpec((1, D), lambda b, j: (0, 0)),
            pl.BlockSpec((1, D), lambda b, j: (0, 0)),
            pl.BlockSpec((1, D), lambda b, j: (0, 0)),
        ],
        out_specs=[
            pl.BlockSpec((BLK, D), lambda b, j: (b * NBLK + j, 0)),
            pl.BlockSpec((1, D), lambda b, j: (0, 0)),
            pl.BlockSpec((1, D), lambda b, j: (0, 0)),
        ],
        out_shape=[
            jax.ShapeDtypeStruct((B * NP, D), jnp.float32),
            jax.ShapeDtypeStruct((1, D), jnp.float32),
            jax.ShapeDtypeStruct((1, D), jnp.float32),
        ],
    )(g, *_sums_pair(g), w_emb_p, gnn_bias[None, :], bn1_gamma[None, :],
      bn1_beta[None, :])


def _sums_pair(g):
    return _sums(g)


def _out_body(outm_ref, t1_ref, t2_ref, gam_ref, bet_ref, ow_ref, ob_ref,
              o_ref):
    mu = t1_ref[...] / TOT
    var = t2_ref[...] / TOT - mu * mu
    h = (outm_ref[...] - mu) / jnp.sqrt(var + 1e-5) * gam_ref[...] + bet_ref[...]
    h = jnp.maximum(h, 0.0)
    o_ref[...] = lax.dot_general(h, ow_ref[...], (((1,), (0,)), ((), ())),
                                 preferred_element_type=jnp.float32) + ob_ref[...]


def _outproj(outm, t1, t2, bn_out_gamma, bn_out_beta, out_W, out_b):
    return pl.pallas_call(
        _out_body,
        grid=(B * NBLK,),
        in_specs=[
            pl.BlockSpec((BLK, D), lambda i: (i, 0)),
            pl.BlockSpec((1, D), lambda i: (0, 0)),
            pl.BlockSpec((1, D), lambda i: (0, 0)),
            pl.BlockSpec((1, D), lambda i: (0, 0)),
            pl.BlockSpec((1, D), lambda i: (0, 0)),
            pl.BlockSpec((D, 1), lambda i: (0, 0)),
            pl.BlockSpec((1, 1), lambda i: (0, 0)),
        ],
        out_specs=pl.BlockSpec((BLK, 1), lambda i: (i, 0)),
        out_shape=jax.ShapeDtypeStruct((B * NP, 1), jnp.float32),
    )(outm, t1, t2, bn_out_gamma[None, :], bn_out_beta[None, :], out_W,
      out_b[:, None])


def kernel(data, W_emb, lin_W, att_i, att_j, att_em_i, att_em_j, gnn_bias,
           bn1_gamma, bn1_beta, bn_out_gamma, bn_out_beta, out_W, out_b):
    w_emb_p = jnp.pad(W_emb, ((0, NP - N), (0, 0)))
    data_p = jnp.pad(data, ((0, 0), (0, NP - N), (0, 0)))
    tk32 = _topk32(w_emb_p)
    xl, s, d = _prep(data_p, lin_W, w_emb_p, att_j, att_em_j, att_i, att_em_i)
    g = _gnn_sc(tk32.reshape(-1), xl.reshape(B * NP, D),
                s.reshape(B, NP), d.reshape(B, NP))
    outm, t1, t2 = _bn1(g, w_emb_p, gnn_bias, bn1_gamma, bn1_beta)
    o = _outproj(outm, t1, t2, bn_out_gamma, bn_out_beta, out_W, out_b)
    return o.reshape(B, NP)[:, :N]


# speedup vs baseline: 1.0259x; 1.0259x over previous
import functools

import jax
import jax.numpy as jnp
from jax import lax
from jax.experimental import pallas as pl
from jax.experimental.pallas import tpu as pltpu
from jax.experimental.pallas import tpu_sc as plsc

N = 10000
NP = 10240
D = 64
F_IN = 10
B = 4
K = 20
KP = 32
BLK = 256
NBLK = NP // BLK
NW = 32
PW = NP // NW
WIN = 16
NWIN = (PW // WIN) * B
ROWS = (K + 1) * WIN
TOT = B * N


def _topk_body(rows_ref, w_ref, nrm_ref, nrmc_ref, out_ref):
    rows = rows_ref[...]
    w = w_ref[...]
    dots = lax.dot_general(rows, w, (((1,), (1,)), ((), ())),
                           preferred_element_type=jnp.float32)
    nrm = nrm_ref[...]
    nrm_rows = nrmc_ref[...]
    cos = dots / (nrm_rows * nrm)
    ci = lax.broadcasted_iota(jnp.int32, (BLK, NP), 1)
    cos = jnp.where(ci < N, cos, -3.0)
    ri = lax.broadcasted_iota(jnp.int32, (BLK, 1), 0) + pl.program_id(0) * BLK
    valid_row = ri < N
    idxs = []
    big = jnp.int32(2**30)
    for _ in range(K):
        m = jnp.max(cos, axis=1, keepdims=True)
        idx = jnp.min(jnp.where(cos >= m, ci, big), axis=1, keepdims=True)
        idxs.append(jnp.where(valid_row, idx, ri))
        cos = jnp.where(ci == idx, -3.0, cos)
    idxs.extend([ri] * (KP - K))
    out_ref[...] = jnp.concatenate(idxs, axis=1)


def _topk32(w_emb_p):
    nrm = jnp.sqrt(jnp.sum(w_emb_p * w_emb_p, axis=1))
    return pl.pallas_call(
        _topk_body,
        grid=(NBLK,),
        in_specs=[
            pl.BlockSpec((BLK, D), lambda i: (i, 0)),
            pl.BlockSpec((NP, D), lambda i: (0, 0)),
            pl.BlockSpec((1, NP), lambda i: (0, 0)),
            pl.BlockSpec((BLK, 1), lambda i: (i, 0)),
        ],
        out_specs=pl.BlockSpec((BLK, KP), lambda i: (i, 0)),
        out_shape=jax.ShapeDtypeStruct((NP, KP), jnp.int32),
    )(w_emb_p, w_emb_p, nrm[None, :], nrm[:, None])


def _prep_body(x_ref, lw_ref, emb_ref, aj_ref, aemj_ref, ai_ref, aemi_ref,
               xl_ref, s_ref, d_ref):
    x = x_ref[0]
    xl = lax.dot_general(x, lw_ref[...], (((1,), (0,)), ((), ())),
                         preferred_element_type=jnp.float32)
    emb = emb_ref[...]
    xl_ref[0] = xl
    s_ref[...] = jnp.sum(xl * aj_ref[...] + emb * aemj_ref[...], axis=1,
                         keepdims=True)
    d_ref[...] = jnp.sum(xl * ai_ref[...] + emb * aemi_ref[...], axis=1,
                         keepdims=True)


def _prep(data_p, lin_W, w_emb_p, att_j, att_em_j, att_i, att_em_i):
    return pl.pallas_call(
        _prep_body,
        grid=(B, NBLK),
        in_specs=[
            pl.BlockSpec((1, BLK, F_IN), lambda b, j: (b, j, 0)),
            pl.BlockSpec((F_IN, D), lambda b, j: (0, 0)),
            pl.BlockSpec((BLK, D), lambda b, j: (j, 0)),
            pl.BlockSpec((1, D), lambda b, j: (0, 0)),
            pl.BlockSpec((1, D), lambda b, j: (0, 0)),
            pl.BlockSpec((1, D), lambda b, j: (0, 0)),
            pl.BlockSpec((1, D), lambda b, j: (0, 0)),
        ],
        out_specs=[
            pl.BlockSpec((1, BLK, D), lambda b, j: (b, j, 0)),
            pl.BlockSpec((BLK, 1), lambda b, j: (b * NBLK + j, 0)),
            pl.BlockSpec((BLK, 1), lambda b, j: (b * NBLK + j, 0)),
        ],
        out_shape=[
            jax.ShapeDtypeStruct((B, NP, D), jnp.float32),
            jax.ShapeDtypeStruct((B * NP, 1), jnp.float32),
            jax.ShapeDtypeStruct((B * NP, 1), jnp.float32),
        ],
    )(data_p, lin_W, w_emb_p, att_j[None, :], att_em_j[None, :],
      att_i[None, :], att_em_i[None, :])


def _gnn_sc_body(tk_hbm, xl_hbm, s_hbm, d_hbm, out_hbm,
                 s_tab, d_tab, tk_v0, tk_v1, gidx0, gidx1, grows0, grows1,
                 wbuf, obuf, sem0, sem1):
    cid = lax.axis_index("c")
    sid = lax.axis_index("s")
    wid = cid * 16 + sid
    base = wid * PW
    pltpu.sync_copy(s_hbm, s_tab)
    pltpu.sync_copy(d_hbm, d_tab)
    lane = lax.broadcasted_iota(jnp.int32, (WIN,), 0)
    wpb = PW // WIN
    par = [(tk_v0, gidx0, grows0, sem0), (tk_v1, gidx1, grows1, sem1)]

    def issue(t, tk_v, gidx, grows, sem):
        b = t // wpb
        node0 = base + (t % wpb) * WIN
        pltpu.sync_copy(tk_hbm.at[pl.ds(node0 * KP, WIN * KP)], tk_v)
        boff = b * NP
        for k in range(K):
            ck = plsc.load_gather(tk_v, [lane * KP + k])
            gidx[pl.ds(k * WIN, WIN)] = ck + boff
        gidx[pl.ds(K * WIN, WIN)] = node0 + lane + boff
        for c in range(3):
            pltpu.async_copy(xl_hbm.at[gidx.at[pl.ds(c * 112, 112)]],
                             grows.at[pl.ds(c * 112, 112)], sem)

    def compute(t, tk_v, gidx, grows, sem):
        b = t // wpb
        node0 = base + (t % wpb) * WIN
        nodes = node0 + lane
        boff = b * NP
        bvec = jnp.full((WIN,), 0, jnp.int32) + b
        d_vec = plsc.load_gather(d_tab, [bvec, nodes])
        s_own = plsc.load_gather(s_tab, [bvec, nodes])
        z = d_vec + s_own
        l_self = jnp.maximum(z, 0.2 * z)
        m = l_self
        cols = []
        for k in range(K):
            ck = plsc.load_gather(tk_v, [lane * KP + k])
            cols.append(ck)
            sk = plsc.load_gather(s_tab, [bvec, ck])
            zk = d_vec + sk
            lk = jnp.maximum(zk, 0.2 * zk)
            lk = jnp.where(ck == nodes, -1e30, lk)
            wbuf[pl.ds(k * WIN, WIN)] = lk
            m = jnp.maximum(m, lk)
        ssum = jnp.exp(l_self - m)
        wbuf[pl.ds(K * WIN, WIN)] = ssum
        for k in range(K):
            ek = jnp.exp(wbuf[pl.ds(k * WIN, WIN)] - m)
            wbuf[pl.ds(k * WIN, WIN)] = ek
            ssum = ssum + ek
        inv = 1.0 / (ssum + 1e-16)
        for c in range(3):
            pltpu.make_async_copy(xl_hbm.at[gidx.at[pl.ds(c * 112, 112)]],
                                  grows.at[pl.ds(c * 112, 112)], sem).wait()
        wv = [wbuf[pl.ds(k * WIN, WIN)] for k in range(K + 1)]

        def dim_step(dd, c):
            dvec = jnp.full((WIN,), 0, jnp.int32) + dd
            acc = wv[0] * plsc.load_gather(grows, [lane, dvec])
            for k in range(1, K + 1):
                acc = acc + wv[k] * plsc.load_gather(grows,
                                                     [k * WIN + lane, dvec])
            plsc.store_scatter(obuf, [lane, dvec], acc * inv)
            return c

        lax.fori_loop(0, D, dim_step, 0)
        pltpu.sync_copy(obuf, out_hbm.at[pl.ds(boff + node0, WIN), :])

    issue(0, *par[0])

    def pair(i, carry):
        t0 = 2 * i
        issue(jnp.minimum(t0 + 1, NWIN - 1), *par[1])
        compute(t0, *par[0])
        issue(jnp.minimum(t0 + 2, NWIN - 1), *par[0])
        compute(t0 + 1, *par[1])
        return carry

    lax.fori_loop(0, NWIN // 2, pair, 0)
    for c in range(3):
        pltpu.make_async_copy(xl_hbm.at[gidx0.at[pl.ds(c * 112, 112)]],
                              grows0.at[pl.ds(c * 112, 112)], sem0).wait()


def _gnn_sc(tk_flat, xl_flat, s2, d2):
    mesh = plsc.VectorSubcoreMesh(core_axis_name="c", subcore_axis_name="s")
    run = pl.kernel(
        _gnn_sc_body,
        out_type=jax.ShapeDtypeStruct((B * NP, D), jnp.float32),
        mesh=mesh,
        compiler_params=pltpu.CompilerParams(needs_layout_passes=False,
                                             use_tc_tiling_on_sc=False),
        scratch_types=[
            pltpu.VMEM((B, NP), jnp.float32),
            pltpu.VMEM((B, NP), jnp.float32),
            pltpu.VMEM((WIN * KP,), jnp.int32),
            pltpu.VMEM((WIN * KP,), jnp.int32),
            pltpu.VMEM((ROWS,), jnp.int32),
            pltpu.VMEM((ROWS,), jnp.int32),
            pltpu.VMEM((ROWS, D), jnp.float32),
            pltpu.VMEM((ROWS, D), jnp.float32),
            pltpu.VMEM((ROWS,), jnp.float32),
            pltpu.VMEM((WIN, D), jnp.float32),
            pltpu.SemaphoreType.DMA,
            pltpu.SemaphoreType.DMA,
        ],
    )
    return run(tk_flat, xl_flat, s2, d2)


def _sums_body(g_ref, s1_ref, s2_ref):
    @pl.when(pl.program_id(0) == 0)
    def _():
        s1_ref[...] = jnp.zeros_like(s1_ref)
        s2_ref[...] = jnp.zeros_like(s2_ref)
    g = g_ref[...]
    s1_ref[...] += jnp.sum(g, axis=0, keepdims=True)
    s2_ref[...] += jnp.sum(g * g, axis=0, keepdims=True)


def _sums(g):
    return pl.pallas_call(
        _sums_body,
        grid=(B * NBLK,),
        in_specs=[pl.BlockSpec((BLK, D), lambda i: (i, 0))],
        out_specs=[pl.BlockSpec((1, D), lambda i: (0, 0)),
                   pl.BlockSpec((1, D), lambda i: (0, 0))],
        out_shape=[jax.ShapeDtypeStruct((1, D), jnp.float32),
                   jax.ShapeDtypeStruct((1, D), jnp.float32)],
    )(g)


def _bn1_body(g_ref, s1_ref, s2_ref, emb_ref, bias_ref, gam_ref, bet_ref,
              outm_ref, t1_ref, t2_ref):
    b = pl.program_id(0)
    j = pl.program_id(1)

    @pl.when(jnp.logical_and(b == 0, j == 0))
    def _():
        t1_ref[...] = jnp.zeros_like(t1_ref)
        t2_ref[...] = jnp.zeros_like(t2_ref)

    bias = bias_ref[...]
    s1 = s1_ref[...] + TOT * bias
    s2 = s2_ref[...] + 2.0 * bias * s1_ref[...] + TOT * bias * bias
    mu = s1 / TOT
    var = s2 / TOT - mu * mu
    x = g_ref[...] + bias
    y = (x - mu) / jnp.sqrt(var + 1e-5) * gam_ref[...] + bet_ref[...]
    y = jnp.maximum(y, 0.0)
    outm = y * emb_ref[...]
    outm_ref[...] = outm
    t1_ref[...] += jnp.sum(outm, axis=0, keepdims=True)
    t2_ref[...] += jnp.sum(outm * outm, axis=0, keepdims=True)


def _bn1(g, w_emb_p, gnn_bias, bn1_gamma, bn1_beta):
    return pl.pallas_call(
        _bn1_body,
        grid=(B, NBLK),
        in_specs=[
            pl.BlockSpec((BLK, D), lambda b, j: (b * NBLK + j, 0)),
            pl.BlockSpec((1, D), lambda b, j: (0, 0)),
            pl.BlockSpec((1, D), lambda b, j: (0, 0)),
            pl.BlockSpec((BLK, D), lambda b, j: (j, 0)),
            pl.BlockSpec((1, D), lambda b, j: (0, 0)),
            pl.BlockSpec((1, D), lambda b, j: (0, 0)),
            pl.BlockSpec((1, D), lambda b, j: (0, 0)),
        ],
        out_specs=[
            pl.BlockSpec((BLK, D), lambda b, j: (b * NBLK + j, 0)),
            pl.BlockSpec((1, D), lambda b, j: (0, 0)),
            pl.BlockSpec((1, D), lambda b, j: (0, 0)),
        ],
        out_shape=[
            jax.ShapeDtypeStruct((B * NP, D), jnp.float32),
            jax.ShapeDtypeStruct((1, D), jnp.float32),
            jax.ShapeDtypeStruct((1, D), jnp.float32),
        ],
    )(g, *_sums_pair(g), w_emb_p, gnn_bias[None, :], bn1_gamma[None, :],
      bn1_beta[None, :])


def _sums_pair(g):
    return _sums(g)


def _out_body(outm_ref, t1_ref, t2_ref, gam_ref, bet_ref, ow_ref, ob_ref,
              o_ref):
    mu = t1_ref[...] / TOT
    var = t2_ref[...] / TOT - mu * mu
    h = (outm_ref[...] - mu) / jnp.sqrt(var + 1e-5) * gam_ref[...] + bet_ref[...]
    h = jnp.maximum(h, 0.0)
    o_ref[...] = lax.dot_general(h, ow_ref[...], (((1,), (0,)), ((), ())),
                                 preferred_element_type=jnp.float32) + ob_ref[...]


def _outproj(outm, t1, t2, bn_out_gamma, bn_out_beta, out_W, out_b):
    return pl.pallas_call(
        _out_body,
        grid=(B * NBLK,),
        in_specs=[
            pl.BlockSpec((BLK, D), lambda i: (i, 0)),
            pl.BlockSpec((1, D), lambda i: (0, 0)),
            pl.BlockSpec((1, D), lambda i: (0, 0)),
            pl.BlockSpec((1, D), lambda i: (0, 0)),
            pl.BlockSpec((1, D), lambda i: (0, 0)),
            pl.BlockSpec((D, 1), lambda i: (0, 0)),
            pl.BlockSpec((1, 1), lambda i: (0, 0)),
        ],
        out_specs=pl.BlockSpec((BLK, 1), lambda i: (i, 0)),
        out_shape=jax.ShapeDtypeStruct((B * NP, 1), jnp.float32),
    )(outm, t1, t2, bn_out_gamma[None, :], bn_out_beta[None, :], out_W,
      out_b[:, None])


def kernel(data, W_emb, lin_W, att_i, att_j, att_em_i, att_em_j, gnn_bias,
           bn1_gamma, bn1_beta, bn_out_gamma, bn_out_beta, out_W, out_b):
    w_emb_p = jnp.pad(W_emb, ((0, NP - N), (0, 0)))
    data_p = jnp.pad(data, ((0, 0), (0, NP - N), (0, 0)))
    tk32 = _topk32(w_emb_p)
    xl, s, d = _prep(data_p, lin_W, w_emb_p, att_j, att_em_j, att_i, att_em_i)
    g = _gnn_sc(tk32.reshape(-1), xl.reshape(B * NP, D),
                s.reshape(B, NP), d.reshape(B, NP))
    outm, t1, t2 = _bn1(g, w_emb_p, gnn_bias, bn1_gamma, bn1_beta)
    o = _outproj(outm, t1, t2, bn_out_gamma, bn_out_beta, out_W, out_b)
    return o.reshape(B, NP)[:, :N]
